# baseline (speedup 1.0000x reference)
import functools

import jax
import jax.numpy as jnp
from jax import lax
from jax.experimental import pallas as pl
from jax.experimental.pallas import tpu as pltpu
from jax.experimental.pallas import tpu_sc as plsc

_G = 16
_SC_SHARE = 12288
_TBLK = 128


def _sc_gather(table_t, idx, n_take):
    D, V = table_t.shape
    info = plsc.get_sparse_core_info()
    nw = info.num_cores * info.num_subcores
    b_per_w = n_take // nw
    n_groups = b_per_w // _G
    mesh = plsc.VectorSubcoreMesh(core_axis_name="c", subcore_axis_name="s")

    @functools.partial(
        pl.kernel,
        mesh=mesh,
        out_type=jax.ShapeDtypeStruct((D, n_take), jnp.float32),
        compiler_params=pltpu.CompilerParams(needs_layout_passes=False),
        scratch_types=[
            pltpu.VMEM((b_per_w,), jnp.int32),
            pltpu.VMEM((_G, D, 128), jnp.float32),
            pltpu.VMEM((D, b_per_w), jnp.float32),
            pltpu.SemaphoreType.DMA,
        ],
    )
    def gather_kernel(tab_hbm, idx_hbm, out_hbm, idx_v, win_v, outb_v, sem):
        wid = lax.axis_index("s") * info.num_cores + lax.axis_index("c")
        base = wid * b_per_w
        pltpu.sync_copy(idx_hbm.at[pl.ds(base, b_per_w)], idx_v)

        lanes = lax.iota(jnp.int32, 16)
        mask127 = jnp.full((16,), 127, jnp.int32)

        def group_step(g, _):
            ivec = idx_v[pl.ds(g * _G, _G)]
            avec = lax.shift_left(
                lax.shift_right_logical(ivec, 7), jnp.full((16,), 7, jnp.int32)
            )
            for u in range(_G):
                off = pl.multiple_of(avec[u], 128)
                pltpu.async_copy(
                    tab_hbm.at[:, pl.ds(off, 128)], win_v.at[u], sem
                )
            for _u in range(_G):
                pltpu.make_async_copy(
                    tab_hbm.at[:, pl.ds(0, 128)], win_v.at[0], sem
                ).wait()

            rvec = lax.bitwise_and(ivec, mask127)
            for j in range(D):
                jvec = jnp.full((16,), j, jnp.int32)
                vec = plsc.load_gather(win_v, [lanes, jvec, rvec])
                outb_v[j, pl.ds(g * _G, _G)] = vec
            return _

        lax.fori_loop(0, n_groups, group_step, None)

        pltpu.sync_copy(outb_v, out_hbm.at[:, pl.ds(base, b_per_w)])

    return gather_kernel(table_t, idx)


def _tc_gather(table_t, idx, n_take):
    D, V = table_t.shape
    n_blocks = n_take // _TBLK

    def body(idx_sref, tab_ref, idx_ref, out_ref, win_ref, sem, prev_sem):
        g = pl.program_id(0)

        @pl.when(g < n_blocks)
        def _fire():
            for u in range(_TBLK):
                i = idx_sref[g * _TBLK + u]
                off = pl.multiple_of((i >> 7) * 128, 128)
                pltpu.make_async_copy(
                    tab_ref.at[:, pl.ds(off, 128)],
                    win_ref.at[g % 2].at[u],
                    sem,
                ).start()

        @pl.when(g > 0)
        def _extract():
            for _u in range(_TBLK):
                pltpu.make_async_copy(
                    tab_ref.at[:, pl.ds(0, 128)],
                    win_ref.at[0].at[0],
                    sem,
                ).wait()
            cvec = lax.bitwise_and(idx_ref[0, 0], 127)
            citer = lax.broadcasted_iota(jnp.int32, (128, 8), 0)
            for q in range(_TBLK // 8):
                w8 = win_ref[(g - 1) % 2, pl.ds(q * 8, 8)]
                w2 = w8.reshape(8 * D, 128)
                c8 = cvec[q * 8:(q + 1) * 8]
                oh = (citer == jnp.broadcast_to(c8[None, :], (128, 8))
                      ).astype(jnp.float32)
                p8 = jax.lax.dot(w2, oh)
                cols = [
                    p8[v * D:(v + 1) * D, v:v + 1] for v in range(8)
                ]
                out_ref[:, pl.ds(q * 8, 8)] = jnp.concatenate(cols, axis=1)

    return pl.pallas_call(
        body,
        grid_spec=pltpu.PrefetchScalarGridSpec(
            num_scalar_prefetch=1,
            grid=(n_blocks + 1,),
            in_specs=[
                pl.BlockSpec(memory_space=pl.ANY),
                pl.BlockSpec(
                    (1, 1, _TBLK),
                    lambda g, *_: (jnp.maximum(g - 1, 0), 0, 0),
                ),
            ],
            out_specs=pl.BlockSpec(
                (D, _TBLK), lambda g, *_: (0, jnp.maximum(g - 1, 0))
            ),
            scratch_shapes=[
                pltpu.VMEM((2, _TBLK, D, 128), jnp.float32),
                pltpu.SemaphoreType.DMA,
                pltpu.SemaphoreType.DMA,
            ],
        ),
        out_shape=jax.ShapeDtypeStruct((D, n_take), jnp.float32),
    )(idx, table_t, idx.reshape(n_blocks, 1, _TBLK))


@jax.jit
def kernel(batch, embedding_weight):
    B = batch.shape[0]
    idx = batch.astype(jnp.int32)
    table_t = embedding_weight.T

    out_sc = _sc_gather(table_t, idx[:_SC_SHARE], _SC_SHARE)
    out_tc = _tc_gather(table_t, idx[_SC_SHARE:], B - _SC_SHARE)
    return jnp.concatenate([out_sc, out_tc], axis=1).T


# speedup vs baseline: 1.0274x; 1.0274x over previous
import functools

import jax
import jax.numpy as jnp
from jax import lax
from jax.experimental import pallas as pl
from jax.experimental.pallas import tpu as pltpu
from jax.experimental.pallas import tpu_sc as plsc

_G = 16
_SC_SHARE = 8192
_TBLK = 128


def _sc_gather(table_t, idx, n_take):
    D, V = table_t.shape
    info = plsc.get_sparse_core_info()
    nw = info.num_cores * info.num_subcores
    b_per_w = n_take // nw
    n_groups = b_per_w // _G
    mesh = plsc.VectorSubcoreMesh(core_axis_name="c", subcore_axis_name="s")

    @functools.partial(
        pl.kernel,
        mesh=mesh,
        out_type=jax.ShapeDtypeStruct((D, n_take), jnp.float32),
        compiler_params=pltpu.CompilerParams(needs_layout_passes=False),
        scratch_types=[
            pltpu.VMEM((b_per_w,), jnp.int32),
            pltpu.VMEM((_G, D, 128), jnp.float32),
            pltpu.VMEM((D, b_per_w), jnp.float32),
            pltpu.SemaphoreType.DMA,
        ],
    )
    def gather_kernel(tab_hbm, idx_hbm, out_hbm, idx_v, win_v, outb_v, sem):
        wid = lax.axis_index("s") * info.num_cores + lax.axis_index("c")
        base = wid * b_per_w
        pltpu.sync_copy(idx_hbm.at[pl.ds(base, b_per_w)], idx_v)

        lanes = lax.iota(jnp.int32, 16)
        mask127 = jnp.full((16,), 127, jnp.int32)

        def group_step(g, _):
            ivec = idx_v[pl.ds(g * _G, _G)]
            avec = lax.shift_left(
                lax.shift_right_logical(ivec, 7), jnp.full((16,), 7, jnp.int32)
            )
            for u in range(_G):
                off = pl.multiple_of(avec[u], 128)
                pltpu.async_copy(
                    tab_hbm.at[:, pl.ds(off, 128)], win_v.at[u], sem
                )
            for _u in range(_G):
                pltpu.make_async_copy(
                    tab_hbm.at[:, pl.ds(0, 128)], win_v.at[0], sem
                ).wait()

            rvec = lax.bitwise_and(ivec, mask127)
            for j in range(D):
                jvec = jnp.full((16,), j, jnp.int32)
                vec = plsc.load_gather(win_v, [lanes, jvec, rvec])
                outb_v[j, pl.ds(g * _G, _G)] = vec
            return _

        lax.fori_loop(0, n_groups, group_step, None)

        pltpu.sync_copy(outb_v, out_hbm.at[:, pl.ds(base, b_per_w)])

    return gather_kernel(table_t, idx)


def _tc_gather(table_t, idx, n_take):
    D, V = table_t.shape
    n_blocks = n_take // _TBLK

    def body(idx_sref, tab_ref, idx_ref, out_ref, win_ref, sem, prev_sem):
        g = pl.program_id(0)

        @pl.when(g < n_blocks)
        def _fire():
            for u in range(_TBLK):
                i = idx_sref[g * _TBLK + u]
                off = pl.multiple_of((i >> 7) * 128, 128)
                pltpu.make_async_copy(
                    tab_ref.at[:, pl.ds(off, 128)],
                    win_ref.at[g % 2].at[u],
                    sem,
                ).start()

        @pl.when(g > 0)
        def _extract():
            for _u in range(_TBLK):
                pltpu.make_async_copy(
                    tab_ref.at[:, pl.ds(0, 128)],
                    win_ref.at[0].at[0],
                    sem,
                ).wait()
            cvec = lax.bitwise_and(idx_ref[0, 0], 127)
            citer = lax.broadcasted_iota(jnp.int32, (128, 8), 0)
            for q in range(_TBLK // 8):
                w8 = win_ref[(g - 1) % 2, pl.ds(q * 8, 8)]
                w2 = w8.reshape(8 * D, 128)
                c8 = cvec[q * 8:(q + 1) * 8]
                oh = (citer == jnp.broadcast_to(c8[None, :], (128, 8))
                      ).astype(jnp.float32)
                p8 = jax.lax.dot(w2, oh)
                cols = [
                    p8[v * D:(v + 1) * D, v:v + 1] for v in range(8)
                ]
                out_ref[:, pl.ds(q * 8, 8)] = jnp.concatenate(cols, axis=1)

    return pl.pallas_call(
        body,
        grid_spec=pltpu.PrefetchScalarGridSpec(
            num_scalar_prefetch=1,
            grid=(n_blocks + 1,),
            in_specs=[
                pl.BlockSpec(memory_space=pl.ANY),
                pl.BlockSpec(
                    (1, 1, _TBLK),
                    lambda g, *_: (jnp.maximum(g - 1, 0), 0, 0),
                ),
            ],
            out_specs=pl.BlockSpec(
                (D, _TBLK), lambda g, *_: (0, jnp.maximum(g - 1, 0))
            ),
            scratch_shapes=[
                pltpu.VMEM((2, _TBLK, D, 128), jnp.float32),
                pltpu.SemaphoreType.DMA,
                pltpu.SemaphoreType.DMA,
            ],
        ),
        out_shape=jax.ShapeDtypeStruct((D, n_take), jnp.float32),
    )(idx, table_t, idx.reshape(n_blocks, 1, _TBLK))


@jax.jit
def kernel(batch, embedding_weight):
    B = batch.shape[0]
    idx = batch.astype(jnp.int32)
    table_t = embedding_weight.T

    out_sc = _sc_gather(table_t, idx[:_SC_SHARE], _SC_SHARE)
    out_tc = _tc_gather(table_t, idx[_SC_SHARE:], B - _SC_SHARE)
    return jnp.concatenate([out_sc, out_tc], axis=1).T
